# baseline (speedup 1.0000x reference)
import functools

import jax
import jax.numpy as jnp
from jax.experimental import pallas as pl
from jax.experimental.pallas import tpu as pltpu

BS = 256
NN = 32
EH = NN // 2
ND = 3
HD = 6
HID = 64
LW = 2 * HID
NL = 4
NS = 2
INV_NORM = 1.0 / 100.0


def _rep_rows(v, r):
    n, f = v.shape
    return jnp.broadcast_to(v[:, None, :], (n, r, f)).reshape(n * r, f)


def _halfpair(v, g):
    f = v.shape[1]
    v3 = v.reshape(g, NN, f)
    lo = v3[:, :EH].reshape(g * EH, f)
    hi = v3[:, EH:].reshape(g * EH, f)
    return jnp.concatenate([lo, hi], axis=1)


def _dup(v):
    return jnp.concatenate([v, v], axis=1)


def _bcast_i(v, g):
    return jnp.broadcast_to(
        v.reshape(g, 1, NN, LW), (g, EH, NN, LW)
    ).reshape(g * EH * NN, LW)


def _bcast_j(vp):
    return _rep_rows(vp, NN)


def _sum_j_raw(e, g):
    return jnp.sum(e.reshape(g, EH, NN, LW), axis=1).reshape(g * NN, LW)


def _sum_j(e, g):
    s = _sum_j_raw(e, g)
    return s[:, :HID] + s[:, HID:]


def _silu(v):
    return v * jnp.tanh(v) + v


def _egnn_body(g,
               t_ref, xh_ref, rep_ref,
               embW_ref, embB_ref, repW_ref, repB_ref, outW_ref, outB_ref,
               eWa_ref, eAt_ref, eW2_ref, eB2_ref,
               nW1_ref, nB1_ref, nW2_ref, nB2_ref, cW3_ref,
               out_ref):
    gn = g * NN
    ones_col = jnp.ones((gn, 1), dtype=jnp.float32)
    lane = jax.lax.broadcasted_iota(jnp.int32, (1, LW), 1) % HID
    lm01 = lane < 16
    lm12 = lane < 32

    xh = xh_ref[...].reshape(gn, ND + HD)
    x0 = [jnp.broadcast_to(xh[:, d:d + 1], (gn, HID)) for d in range(ND)]
    x = list(x0)

    h_time = _rep_rows(t_ref[...], NN)
    h7 = jnp.concatenate([xh[:, ND:], h_time], axis=1)
    h = jnp.dot(h7, embW_ref[...], preferred_element_type=jnp.float32)
    h = h + embB_ref[...]
    rep_h = jnp.dot(rep_ref[...], repW_ref[...],
                    preferred_element_type=jnp.float32) + repB_ref[...]
    h = h + _rep_rows(rep_h, NN)

    diff0 = [_bcast_i(_dup(x0[d]), g) - _bcast_j(_halfpair(x0[d], g))
             for d in range(ND)]
    d2_0 = diff0[0] * diff0[0] + diff0[1] * diff0[1] + diff0[2] * diff0[2]

    for blk in range(NL):
        if blk == 0:
            diff, radial = diff0, d2_0
        else:
            diff = [_bcast_i(_dup(x[d]), g) - _bcast_j(_halfpair(x[d], g))
                    for d in range(ND)]
            radial = (diff[0] * diff[0] + diff[1] * diff[1]
                      + diff[2] * diff[2])
        norm = jnp.sqrt(radial + 1e-8)
        inv_n = 1.0 / (norm + 1.0)
        diff_p = jnp.where(lm01, diff[0], jnp.where(lm12, diff[1], diff[2]))

        def edge_mlp(m, h_cur, rad):
            ab = jnp.dot(jnp.concatenate([h_cur, ones_col], axis=1),
                         eWa_ref[m], preferred_element_type=jnp.float32)
            if rad is d2_0:
                attr = d2_0 * (eAt_ref[m, 0:1, :] + eAt_ref[m, 1:2, :])
            else:
                attr = (rad * eAt_ref[m, 0:1, :]
                        + d2_0 * eAt_ref[m, 1:2, :])
            pre1 = (_bcast_i(_dup(ab[:, :HID]), g)
                    + _bcast_j(_halfpair(ab[:, HID:], g))
                    + attr)
            t1 = _silu(pre1)
            pre2 = jnp.dot(t1, eW2_ref[m],
                           preferred_element_type=jnp.float32) + eB2_ref[m]
            return _silu(pre2)

        for s in range(NS):
            m = blk * (NS + 1) + s
            mij = edge_mlp(m, h, radial)
            agg = _sum_j(mij, g)
            nin = jnp.concatenate([h, agg], axis=1)
            n1 = _silu(jnp.dot(nin, nW1_ref[blk * NS + s],
                               preferred_element_type=jnp.float32)
                       + nB1_ref[blk * NS + s])
            h = h + jnp.dot(n1, nW2_ref[blk * NS + s],
                            preferred_element_type=jnp.float32) \
                  + nB2_ref[blk * NS + s]

        m = blk * (NS + 1) + NS
        t2 = edge_mlp(m, h, radial)
        phi = jnp.dot(t2, cW3_ref[blk], preferred_element_type=jnp.float32)
        s = _sum_j_raw(diff_p * (phi * inv_n), g)
        for d in range(ND):
            col = s[:, 16 * d:16 * d + 1] + s[:, HID + 16 * d:HID + 16 * d + 1]
            x[d] = x[d] + jnp.broadcast_to(col, (gn, HID))

    h_out = jnp.dot(h, outW_ref[...],
                    preferred_element_type=jnp.float32) + outB_ref[...]

    vel_cols = []
    for d in range(ND):
        vd = x[d] - x0[d]
        mean = jnp.mean(vd.reshape(g, NN, HID), axis=1, keepdims=True)
        vd = (vd.reshape(g, NN, HID) - mean).reshape(gn, HID)
        vel_cols.append(vd[:, 0:1])
    out = jnp.concatenate(vel_cols + [h_out[:, :HD]], axis=1)
    out_ref[...] = out.reshape(g, NN, ND + HD)


def _blockdiag(w):
    z = jnp.zeros_like(w)
    return jnp.concatenate([jnp.concatenate([w, z], axis=1),
                            jnp.concatenate([z, w], axis=1)], axis=0)


def kernel(t, xh, node_mask, edge_mask, rep, params):
    del node_mask, edge_mask
    g = 16
    grid = (BS // g,)

    eWa, eAt, eW2, eB2 = [], [], [], []
    nW1, nB1, nW2, nB2, cW3 = [], [], [], [], []

    def pack_edge_mlp(W1, b1, W2, b2):
        wa = jnp.concatenate([W1[:HID], W1[HID:2 * HID]], axis=1)
        bias_row = jnp.concatenate([b1, jnp.zeros_like(b1)])[None, :]
        eWa.append(0.5 * jnp.concatenate([wa, bias_row], axis=0))
        eAt.append(0.5 * _dup(W1[2 * HID:]))
        eW2.append(0.5 * _blockdiag(W2))
        eB2.append(0.5 * _dup(b2[None, :]))

    for blk in params["blocks"]:
        for gc in blk["gcls"]:
            pack_edge_mlp(gc["eW1"], gc["eb1"], gc["eW2"], gc["eb2"])
            nW1.append(jnp.concatenate(
                [0.5 * gc["nW1"][:HID],
                 gc["nW1"][HID:] * (0.5 * INV_NORM)], axis=0))
            nB1.append(0.5 * gc["nb1"][None, :])
            nW2.append(gc["nW2"])
            nB2.append(gc["nb2"][None, :])
        pack_edge_mlp(blk["cW1"], blk["cb1"], blk["cW2"], blk["cb2"])
        cW3.append(_blockdiag(
            jnp.broadcast_to(blk["cW3"] * INV_NORM, (HID, HID))))
    eWa = jnp.stack(eWa); eAt = jnp.stack(eAt)
    eW2 = jnp.stack(eW2); eB2 = jnp.stack(eB2)
    nW1 = jnp.stack(nW1); nB1 = jnp.stack(nB1)
    nW2 = jnp.stack(nW2); nB2 = jnp.stack(nB2)
    cW3 = jnp.stack(cW3)

    embB = params["emb_b"][None, :]
    repB = params["rep_b"][None, :]
    outB = params["out_b"][None, :]

    def full(a):
        return pl.BlockSpec(a.shape, lambda i: (0,) * a.ndim)

    weights = [params["emb_W"], embB, params["rep_W"], repB,
               params["out_W"], outB,
               eWa, eAt, eW2, eB2, nW1, nB1, nW2, nB2, cW3]

    out = pl.pallas_call(
        functools.partial(_egnn_body, g),
        grid=grid,
        in_specs=[
            pl.BlockSpec((g, 1), lambda i: (i, 0)),
            pl.BlockSpec((g, NN, ND + HD), lambda i: (i, 0, 0)),
            pl.BlockSpec((g, rep.shape[1]), lambda i: (i, 0)),
        ] + [full(w) for w in weights],
        out_specs=pl.BlockSpec((g, NN, ND + HD), lambda i: (i, 0, 0)),
        out_shape=jax.ShapeDtypeStruct((BS, NN, ND + HD), jnp.float32),
        compiler_params=pltpu.CompilerParams(
            dimension_semantics=("parallel",),
        ),
    )(t, xh, rep, *weights)
    return out


# speedup vs baseline: 1.1619x; 1.1619x over previous
import functools

import jax
import jax.numpy as jnp
from jax.experimental import pallas as pl
from jax.experimental.pallas import tpu as pltpu

BS = 256
NN = 32
EH = NN // 2
ND = 3
HD = 6
HID = 64
LW = 2 * HID
NL = 4
NS = 2
INV_NORM = 1.0 / 100.0


def _rep_rows(v, r):
    n, f = v.shape
    return jnp.broadcast_to(v[:, None, :], (n, r, f)).reshape(n * r, f)


def _halfpair(v, g):
    f = v.shape[1]
    v3 = v.reshape(g, NN, f)
    lo = v3[:, :EH].reshape(g * EH, f)
    hi = v3[:, EH:].reshape(g * EH, f)
    return jnp.concatenate([lo, hi], axis=1)


def _dup(v):
    return jnp.concatenate([v, v], axis=1)


def _i4(v, g):
    return v.reshape(g, 1, NN, LW)


def _j4(vp, g):
    return vp.reshape(g, EH, 1, LW)


def _sum_j_raw(e4, g):
    return jnp.sum(e4, axis=1).reshape(g * NN, LW)


def _sum_j(e4, g):
    s = _sum_j_raw(e4, g)
    return s[:, :HID] + s[:, HID:]


def _silu(v):
    return v * jnp.tanh(v) + v


def _egnn_body(g,
               t_ref, xh_ref, rep_ref,
               embW_ref, embB_ref, repW_ref, repB_ref, outW_ref, outB_ref,
               eWa_ref, eAt_ref, eW2_ref, eB2_ref,
               nW1_ref, nB1_ref, nW2_ref, nB2_ref, cW3_ref,
               out_ref):
    gn = g * NN
    e2 = g * EH * NN
    ones_col = jnp.ones((gn, 1), dtype=jnp.float32)
    lane = jax.lax.broadcasted_iota(jnp.int32, (1, 1, 1, LW), 3) % HID
    lm01 = lane < 16
    lm12 = lane < 32

    xh = xh_ref[...].reshape(gn, ND + HD)
    x0 = [jnp.broadcast_to(xh[:, d:d + 1], (gn, HID)) for d in range(ND)]
    x = list(x0)

    h_time = _rep_rows(t_ref[...], NN)
    h7 = jnp.concatenate([xh[:, ND:], h_time], axis=1)
    h = jnp.dot(h7, embW_ref[...], preferred_element_type=jnp.float32)
    h = h + embB_ref[...]
    rep_h = jnp.dot(rep_ref[...], repW_ref[...],
                    preferred_element_type=jnp.float32) + repB_ref[...]
    h = h + _rep_rows(rep_h, NN)

    diff0 = [_i4(_dup(x0[d]), g) - _j4(_halfpair(x0[d], g), g)
             for d in range(ND)]
    d2_0 = diff0[0] * diff0[0] + diff0[1] * diff0[1] + diff0[2] * diff0[2]

    for blk in range(NL):
        if blk == 0:
            diff, radial = diff0, d2_0
        else:
            diff = [_i4(_dup(x[d]), g) - _j4(_halfpair(x[d], g), g)
                    for d in range(ND)]
            radial = (diff[0] * diff[0] + diff[1] * diff[1]
                      + diff[2] * diff[2])
        norm = jnp.sqrt(radial + 1e-8)
        inv_n = 1.0 / (norm + 1.0)
        diff_p = jnp.where(lm01, diff[0], jnp.where(lm12, diff[1], diff[2]))

        def edge_mlp(m, h_cur, rad):
            ab = jnp.dot(jnp.concatenate([h_cur, ones_col], axis=1),
                         eWa_ref[m], preferred_element_type=jnp.float32)
            w0 = eAt_ref[m, 0, :].reshape(1, 1, 1, LW)
            w1 = eAt_ref[m, 1, :].reshape(1, 1, 1, LW)
            if rad is d2_0:
                attr = d2_0 * (w0 + w1)
            else:
                attr = rad * w0 + d2_0 * w1
            pre1 = (_i4(_dup(ab[:, :HID]), g)
                    + _j4(_halfpair(ab[:, HID:], g), g)
                    + attr)
            t1 = _silu(pre1).reshape(e2, LW)
            pre2 = jnp.dot(t1, eW2_ref[m],
                           preferred_element_type=jnp.float32) + eB2_ref[m]
            return _silu(pre2).reshape(g, EH, NN, LW)

        for s in range(NS):
            m = blk * (NS + 1) + s
            mij = edge_mlp(m, h, radial)
            agg = _sum_j(mij, g)
            nin = jnp.concatenate([h, agg], axis=1)
            n1 = _silu(jnp.dot(nin, nW1_ref[blk * NS + s],
                               preferred_element_type=jnp.float32)
                       + nB1_ref[blk * NS + s])
            h = h + jnp.dot(n1, nW2_ref[blk * NS + s],
                            preferred_element_type=jnp.float32) \
                  + nB2_ref[blk * NS + s]

        m = blk * (NS + 1) + NS
        t2 = edge_mlp(m, h, radial)
        phi = jnp.dot(t2.reshape(e2, LW), cW3_ref[blk],
                      preferred_element_type=jnp.float32
                      ).reshape(g, EH, NN, LW)
        s = _sum_j_raw(diff_p * (phi * inv_n), g)
        for d in range(ND):
            col = s[:, 16 * d:16 * d + 1] + s[:, HID + 16 * d:HID + 16 * d + 1]
            x[d] = x[d] + jnp.broadcast_to(col, (gn, HID))

    h_out = jnp.dot(h, outW_ref[...],
                    preferred_element_type=jnp.float32) + outB_ref[...]

    vel_cols = []
    for d in range(ND):
        vd = x[d] - x0[d]
        mean = jnp.mean(vd.reshape(g, NN, HID), axis=1, keepdims=True)
        vd = (vd.reshape(g, NN, HID) - mean).reshape(gn, HID)
        vel_cols.append(vd[:, 0:1])
    out = jnp.concatenate(vel_cols + [h_out[:, :HD]], axis=1)
    out_ref[...] = out.reshape(g, NN, ND + HD)


def _blockdiag(w):
    z = jnp.zeros_like(w)
    return jnp.concatenate([jnp.concatenate([w, z], axis=1),
                            jnp.concatenate([z, w], axis=1)], axis=0)


def kernel(t, xh, node_mask, edge_mask, rep, params):
    del node_mask, edge_mask
    g = 8
    grid = (BS // g,)

    eWa, eAt, eW2, eB2 = [], [], [], []
    nW1, nB1, nW2, nB2, cW3 = [], [], [], [], []

    def pack_edge_mlp(W1, b1, W2, b2):
        wa = jnp.concatenate([W1[:HID], W1[HID:2 * HID]], axis=1)
        bias_row = jnp.concatenate([b1, jnp.zeros_like(b1)])[None, :]
        eWa.append(0.5 * jnp.concatenate([wa, bias_row], axis=0))
        eAt.append(0.5 * _dup(W1[2 * HID:]))
        eW2.append(0.5 * _blockdiag(W2))
        eB2.append(0.5 * _dup(b2[None, :]))

    for blk in params["blocks"]:
        for gc in blk["gcls"]:
            pack_edge_mlp(gc["eW1"], gc["eb1"], gc["eW2"], gc["eb2"])
            nW1.append(jnp.concatenate(
                [0.5 * gc["nW1"][:HID],
                 gc["nW1"][HID:] * (0.5 * INV_NORM)], axis=0))
            nB1.append(0.5 * gc["nb1"][None, :])
            nW2.append(gc["nW2"])
            nB2.append(gc["nb2"][None, :])
        pack_edge_mlp(blk["cW1"], blk["cb1"], blk["cW2"], blk["cb2"])
        cW3.append(_blockdiag(
            jnp.broadcast_to(blk["cW3"] * INV_NORM, (HID, HID))))
    eWa = jnp.stack(eWa); eAt = jnp.stack(eAt)
    eW2 = jnp.stack(eW2); eB2 = jnp.stack(eB2)
    nW1 = jnp.stack(nW1); nB1 = jnp.stack(nB1)
    nW2 = jnp.stack(nW2); nB2 = jnp.stack(nB2)
    cW3 = jnp.stack(cW3)

    embB = params["emb_b"][None, :]
    repB = params["rep_b"][None, :]
    outB = params["out_b"][None, :]

    def full(a):
        return pl.BlockSpec(a.shape, lambda i: (0,) * a.ndim)

    weights = [params["emb_W"], embB, params["rep_W"], repB,
               params["out_W"], outB,
               eWa, eAt, eW2, eB2, nW1, nB1, nW2, nB2, cW3]

    out = pl.pallas_call(
        functools.partial(_egnn_body, g),
        grid=grid,
        in_specs=[
            pl.BlockSpec((g, 1), lambda i: (i, 0)),
            pl.BlockSpec((g, NN, ND + HD), lambda i: (i, 0, 0)),
            pl.BlockSpec((g, rep.shape[1]), lambda i: (i, 0)),
        ] + [full(w) for w in weights],
        out_specs=pl.BlockSpec((g, NN, ND + HD), lambda i: (i, 0, 0)),
        out_shape=jax.ShapeDtypeStruct((BS, NN, ND + HD), jnp.float32),
        compiler_params=pltpu.CompilerParams(
            dimension_semantics=("parallel",),
        ),
    )(t, xh, rep, *weights)
    return out


# speedup vs baseline: 1.1620x; 1.0001x over previous
import functools

import jax
import jax.numpy as jnp
from jax.experimental import pallas as pl
from jax.experimental.pallas import tpu as pltpu

BS = 256
NN = 32
EH = NN // 2
ND = 3
HD = 6
HID = 64
LW = 2 * HID
NL = 4
NS = 2
INV_NORM = 1.0 / 100.0


def _rep_rows(v, r):
    n, f = v.shape
    return jnp.broadcast_to(v[:, None, :], (n, r, f)).reshape(n * r, f)


def _halfpair(v, g):
    f = v.shape[1]
    v3 = v.reshape(g, NN, f)
    lo = v3[:, :EH].reshape(g * EH, f)
    hi = v3[:, EH:].reshape(g * EH, f)
    return jnp.concatenate([lo, hi], axis=1)


def _dup(v):
    return jnp.concatenate([v, v], axis=1)


def _i4(v, g):
    return v.reshape(g, 1, NN, LW)


def _j4(vp, g):
    return vp.reshape(g, EH, 1, LW)


def _sum_j_raw(e4, g):
    v = e4
    while v.shape[1] > 1:
        half = v.shape[1] // 2
        v = v[:, :half] + v[:, half:]
    return v.reshape(g * NN, LW)


def _sum_j(e4, g):
    s = _sum_j_raw(e4, g)
    return s[:, :HID] + s[:, HID:]


def _silu(v):
    return v * jnp.tanh(v) + v


def _egnn_body(g,
               t_ref, xh_ref, rep_ref,
               embW_ref, embB_ref, repW_ref, repB_ref, outW_ref, outB_ref,
               eWa_ref, eAt_ref, eW2_ref, eB2_ref,
               nW1_ref, nB1_ref, nW2_ref, nB2_ref, cW3_ref,
               out_ref):
    gn = g * NN
    e2 = g * EH * NN
    ones_col = jnp.ones((gn, 1), dtype=jnp.float32)
    lane = jax.lax.broadcasted_iota(jnp.int32, (1, 1, 1, LW), 3) % HID
    lm01 = lane < 16
    lm12 = lane < 32

    xh = xh_ref[...].reshape(gn, ND + HD)
    x0 = [jnp.broadcast_to(xh[:, d:d + 1], (gn, HID)) for d in range(ND)]
    x = list(x0)

    h_time = _rep_rows(t_ref[...], NN)
    h7 = jnp.concatenate([xh[:, ND:], h_time], axis=1)
    h = jnp.dot(h7, embW_ref[...], preferred_element_type=jnp.float32)
    h = h + embB_ref[...]
    rep_h = jnp.dot(rep_ref[...], repW_ref[...],
                    preferred_element_type=jnp.float32) + repB_ref[...]
    h = h + _rep_rows(rep_h, NN)

    diff0 = [_i4(_dup(x0[d]), g) - _j4(_halfpair(x0[d], g), g)
             for d in range(ND)]
    d2_0 = diff0[0] * diff0[0] + diff0[1] * diff0[1] + diff0[2] * diff0[2]

    for blk in range(NL):
        if blk == 0:
            diff, radial = diff0, d2_0
        else:
            diff = [_i4(_dup(x[d]), g) - _j4(_halfpair(x[d], g), g)
                    for d in range(ND)]
            radial = (diff[0] * diff[0] + diff[1] * diff[1]
                      + diff[2] * diff[2])
        norm = jnp.sqrt(radial + 1e-8)
        inv_n = 1.0 / (norm + 1.0)
        diff_p = jnp.where(lm01, diff[0], jnp.where(lm12, diff[1], diff[2]))

        def edge_mlp(m, h_cur, rad):
            ab = jnp.dot(jnp.concatenate([h_cur, ones_col], axis=1),
                         eWa_ref[m], preferred_element_type=jnp.float32)
            w0 = eAt_ref[m, 0, :].reshape(1, 1, 1, LW)
            w1 = eAt_ref[m, 1, :].reshape(1, 1, 1, LW)
            if rad is d2_0:
                attr = d2_0 * (w0 + w1)
            else:
                attr = rad * w0 + d2_0 * w1
            pre1 = (_i4(_dup(ab[:, :HID]), g)
                    + _j4(_halfpair(ab[:, HID:], g), g)
                    + attr)
            t1 = _silu(pre1).reshape(e2, LW)
            pre2 = jnp.dot(t1, eW2_ref[m],
                           preferred_element_type=jnp.float32) + eB2_ref[m]
            return _silu(pre2).reshape(g, EH, NN, LW)

        for s in range(NS):
            m = blk * (NS + 1) + s
            mij = edge_mlp(m, h, radial)
            agg = _sum_j(mij, g)
            nin = jnp.concatenate([h, agg], axis=1)
            n1 = _silu(jnp.dot(nin, nW1_ref[blk * NS + s],
                               preferred_element_type=jnp.float32)
                       + nB1_ref[blk * NS + s])
            h = h + jnp.dot(n1, nW2_ref[blk * NS + s],
                            preferred_element_type=jnp.float32) \
                  + nB2_ref[blk * NS + s]

        m = blk * (NS + 1) + NS
        t2 = edge_mlp(m, h, radial)
        phi = jnp.dot(t2.reshape(e2, LW), cW3_ref[blk],
                      preferred_element_type=jnp.float32
                      ).reshape(g, EH, NN, LW)
        s = _sum_j_raw(diff_p * (phi * inv_n), g)
        for d in range(ND):
            col = s[:, 16 * d:16 * d + 1] + s[:, HID + 16 * d:HID + 16 * d + 1]
            x[d] = x[d] + jnp.broadcast_to(col, (gn, HID))

    h_out = jnp.dot(h, outW_ref[...],
                    preferred_element_type=jnp.float32) + outB_ref[...]

    vel_cols = []
    for d in range(ND):
        vd = x[d] - x0[d]
        mean = jnp.mean(vd.reshape(g, NN, HID), axis=1, keepdims=True)
        vd = (vd.reshape(g, NN, HID) - mean).reshape(gn, HID)
        vel_cols.append(vd[:, 0:1])
    out = jnp.concatenate(vel_cols + [h_out[:, :HD]], axis=1)
    out_ref[...] = out.reshape(g, NN, ND + HD)


def _blockdiag(w):
    z = jnp.zeros_like(w)
    return jnp.concatenate([jnp.concatenate([w, z], axis=1),
                            jnp.concatenate([z, w], axis=1)], axis=0)


def kernel(t, xh, node_mask, edge_mask, rep, params):
    del node_mask, edge_mask
    g = 8
    grid = (BS // g,)

    eWa, eAt, eW2, eB2 = [], [], [], []
    nW1, nB1, nW2, nB2, cW3 = [], [], [], [], []

    def pack_edge_mlp(W1, b1, W2, b2):
        wa = jnp.concatenate([W1[:HID], W1[HID:2 * HID]], axis=1)
        bias_row = jnp.concatenate([b1, jnp.zeros_like(b1)])[None, :]
        eWa.append(0.5 * jnp.concatenate([wa, bias_row], axis=0))
        eAt.append(0.5 * _dup(W1[2 * HID:]))
        eW2.append(0.5 * _blockdiag(W2))
        eB2.append(0.5 * _dup(b2[None, :]))

    for blk in params["blocks"]:
        for gc in blk["gcls"]:
            pack_edge_mlp(gc["eW1"], gc["eb1"], gc["eW2"], gc["eb2"])
            nW1.append(jnp.concatenate(
                [0.5 * gc["nW1"][:HID],
                 gc["nW1"][HID:] * (0.5 * INV_NORM)], axis=0))
            nB1.append(0.5 * gc["nb1"][None, :])
            nW2.append(gc["nW2"])
            nB2.append(gc["nb2"][None, :])
        pack_edge_mlp(blk["cW1"], blk["cb1"], blk["cW2"], blk["cb2"])
        cW3.append(_blockdiag(
            jnp.broadcast_to(blk["cW3"] * INV_NORM, (HID, HID))))
    eWa = jnp.stack(eWa); eAt = jnp.stack(eAt)
    eW2 = jnp.stack(eW2); eB2 = jnp.stack(eB2)
    nW1 = jnp.stack(nW1); nB1 = jnp.stack(nB1)
    nW2 = jnp.stack(nW2); nB2 = jnp.stack(nB2)
    cW3 = jnp.stack(cW3)

    embB = params["emb_b"][None, :]
    repB = params["rep_b"][None, :]
    outB = params["out_b"][None, :]

    def full(a):
        return pl.BlockSpec(a.shape, lambda i: (0,) * a.ndim)

    weights = [params["emb_W"], embB, params["rep_W"], repB,
               params["out_W"], outB,
               eWa, eAt, eW2, eB2, nW1, nB1, nW2, nB2, cW3]

    out = pl.pallas_call(
        functools.partial(_egnn_body, g),
        grid=grid,
        in_specs=[
            pl.BlockSpec((g, 1), lambda i: (i, 0)),
            pl.BlockSpec((g, NN, ND + HD), lambda i: (i, 0, 0)),
            pl.BlockSpec((g, rep.shape[1]), lambda i: (i, 0)),
        ] + [full(w) for w in weights],
        out_specs=pl.BlockSpec((g, NN, ND + HD), lambda i: (i, 0, 0)),
        out_shape=jax.ShapeDtypeStruct((BS, NN, ND + HD), jnp.float32),
        compiler_params=pltpu.CompilerParams(
            dimension_semantics=("parallel",),
        ),
    )(t, xh, rep, *weights)
    return out
